# baseline (speedup 1.0000x reference)
import functools

import jax
import jax.numpy as jnp
from jax import lax
from jax.experimental import pallas as pl
from jax.experimental.pallas import tpu as pltpu
from jax.experimental.pallas import tpu_sc as plsc

B = 16384
D = 64
H = 128
OUT = 64

NC = 2
NS = 16
NW = NC * NS
B_PER_W = B // NW

K_INFLIGHT = 32


def _sc_gather_both(user_table, product_table, user_ids, product_ids):
  mesh = plsc.VectorSubcoreMesh(core_axis_name="c", subcore_axis_name="s")

  @functools.partial(
      pl.kernel,
      mesh=mesh,
      compiler_params=pltpu.CompilerParams(disable_bounds_checks=True),
      out_type=(
          jax.ShapeDtypeStruct((B, D), jnp.float32),
          jax.ShapeDtypeStruct((B, D), jnp.float32),
      ),
      scratch_types=[
          pltpu.VMEM((B_PER_W,), jnp.int32),
          pltpu.VMEM((B_PER_W,), jnp.int32),
          pltpu.SemaphoreType.DMA,
      ],
  )
  def k(utab_hbm, ptab_hbm, uid_hbm, pid_hbm, uout_hbm, pout_hbm,
        uidx_s, pidx_s, sem):
    wid = lax.axis_index("s") * NC + lax.axis_index("c")
    base = wid * B_PER_W
    pltpu.sync_copy(uid_hbm.at[pl.ds(base, B_PER_W)], uidx_s)
    pltpu.sync_copy(pid_hbm.at[pl.ds(base, B_PER_W)], pidx_s)

    def drain_one():
      pltpu.make_async_copy(
          utab_hbm.at[pl.ds(0, 1)],
          uout_hbm.at[pl.ds(base, 1)], sem).wait()

    nv = B_PER_W // 16

    @pl.loop(0, nv)
    def _body(i):
      uv = uidx_s[pl.ds(i * 16, 16)]
      pv = pidx_s[pl.ds(i * 16, 16)]
      for j in range(16):
        pltpu.async_copy(
            utab_hbm.at[pl.ds(uv[j], 1)],
            uout_hbm.at[pl.ds(base + i * 16 + j, 1)], sem)
        pltpu.async_copy(
            ptab_hbm.at[pl.ds(pv[j], 1)],
            pout_hbm.at[pl.ds(base + i * 16 + j, 1)], sem)

      @pl.when(i >= 1)
      def _drain():
        for _ in range(32):
          drain_one()

    @pl.loop(0, 32)
    def _epilogue(i):
      drain_one()

  return k(user_table, product_table, user_ids, product_ids)


BM = 2048


def _tc_mlp_body(u_ref, p_ref, wq1, bq1, wq2, bq2,
                 wc1, bc1, wc2, bc2, q_ref, c_ref):
  q = jnp.maximum(
      jnp.dot(u_ref[...], wq1[...], preferred_element_type=jnp.float32)
      + bq1[...], 0.0)
  q_ref[...] = jnp.maximum(
      jnp.dot(q, wq2[...], preferred_element_type=jnp.float32)
      + bq2[...], 0.0)
  c = jnp.maximum(
      jnp.dot(p_ref[...], wc1[...], preferred_element_type=jnp.float32)
      + bc1[...], 0.0)
  c_ref[...] = jnp.maximum(
      jnp.dot(c, wc2[...], preferred_element_type=jnp.float32)
      + bc2[...], 0.0)


def _tc_towers(pooled_u, pooled_p, Wq1, bq1, Wq2, bq2, Wc1, bc1, Wc2, bc2):
  full = lambda shape: pl.BlockSpec(shape, lambda i: (0, 0))
  return pl.pallas_call(
      _tc_mlp_body,
      grid=(B // BM,),
      in_specs=[
          pl.BlockSpec((BM, D), lambda i: (i, 0)),
          pl.BlockSpec((BM, D), lambda i: (i, 0)),
          full((D, H)), full((1, H)), full((H, OUT)), full((1, OUT)),
          full((D, H)), full((1, H)), full((H, OUT)), full((1, OUT)),
      ],
      out_specs=[
          pl.BlockSpec((BM, OUT), lambda i: (i, 0)),
          pl.BlockSpec((BM, OUT), lambda i: (i, 0)),
      ],
      out_shape=[
          jax.ShapeDtypeStruct((B, OUT), jnp.float32),
          jax.ShapeDtypeStruct((B, OUT), jnp.float32),
      ],
  )(pooled_u, pooled_p,
    Wq1, bq1.reshape(1, H), Wq2, bq2.reshape(1, OUT),
    Wc1, bc1.reshape(1, H), Wc2, bc2.reshape(1, OUT))


@jax.jit
def kernel(user_ids, product_ids, user_table, product_table,
           Wq1, bq1, Wq2, bq2, Wc1, bc1, Wc2, bc2):
  pooled_u, pooled_p = _sc_gather_both(
      user_table, product_table, user_ids, product_ids)
  q, c = _tc_towers(pooled_u, pooled_p,
                    Wq1, bq1, Wq2, bq2, Wc1, bc1, Wc2, bc2)
  return (q, c)


# speedup vs baseline: 2.0490x; 2.0490x over previous
import functools

import jax
import jax.numpy as jnp
from jax import lax
from jax.experimental import pallas as pl
from jax.experimental.pallas import tpu as pltpu
from jax.experimental.pallas import tpu_sc as plsc

B = 16384
D = 64
H = 128
OUT = 64

NC = 2
NS = 16
NW = NC * NS
B_PER_W = B // NW

K_INFLIGHT = 32


def _sc_gather_both(user_table, product_table, user_ids, product_ids):
  mesh = plsc.VectorSubcoreMesh(core_axis_name="c", subcore_axis_name="s")

  @functools.partial(
      pl.kernel,
      mesh=mesh,
      compiler_params=pltpu.CompilerParams(disable_bounds_checks=True),
      out_type=(
          jax.ShapeDtypeStruct((B, D), jnp.float32),
          jax.ShapeDtypeStruct((B, D), jnp.float32),
      ),
      scratch_types=[
          pltpu.VMEM((B_PER_W,), jnp.int32),
          pltpu.VMEM((B_PER_W,), jnp.int32),
          pltpu.VMEM((B_PER_W, D), jnp.float32),
          pltpu.SemaphoreType.DMA,
      ],
  )
  def k(utab_hbm, ptab_hbm, uid_hbm, pid_hbm, uout_hbm, pout_hbm,
        uidx_s, pidx_s, rows_v, sem):
    wid = lax.axis_index("s") * NC + lax.axis_index("c")
    base = wid * B_PER_W
    pltpu.sync_copy(uid_hbm.at[pl.ds(base, B_PER_W)], uidx_s)
    pltpu.sync_copy(pid_hbm.at[pl.ds(base, B_PER_W)], pidx_s)

    nv = B_PER_W // 16

    def gather_one(tab_hbm, idx_s, out_hbm):
      def drain_one():
        pltpu.make_async_copy(
            tab_hbm.at[pl.ds(0, 1)],
            rows_v.at[pl.ds(0, 1)], sem).wait()

      @pl.loop(0, nv)
      def _body(i):
        v = idx_s[pl.ds(i * 16, 16)]
        for j in range(16):
          pltpu.async_copy(
              tab_hbm.at[pl.ds(v[j], 1)],
              rows_v.at[pl.ds(i * 16 + j, 1)], sem)

        @pl.when(i >= 1)
        def _drain():
          for _ in range(16):
            drain_one()

      @pl.loop(0, 16)
      def _epilogue(i):
        drain_one()

      pltpu.sync_copy(rows_v, out_hbm.at[pl.ds(base, B_PER_W)])

    gather_one(utab_hbm, uidx_s, uout_hbm)
    gather_one(ptab_hbm, pidx_s, pout_hbm)

  return k(user_table, product_table, user_ids, product_ids)


BM = 2048


def _tc_mlp_body(u_ref, p_ref, wq1, bq1, wq2, bq2,
                 wc1, bc1, wc2, bc2, q_ref, c_ref):
  q = jnp.maximum(
      jnp.dot(u_ref[...], wq1[...], preferred_element_type=jnp.float32)
      + bq1[...], 0.0)
  q_ref[...] = jnp.maximum(
      jnp.dot(q, wq2[...], preferred_element_type=jnp.float32)
      + bq2[...], 0.0)
  c = jnp.maximum(
      jnp.dot(p_ref[...], wc1[...], preferred_element_type=jnp.float32)
      + bc1[...], 0.0)
  c_ref[...] = jnp.maximum(
      jnp.dot(c, wc2[...], preferred_element_type=jnp.float32)
      + bc2[...], 0.0)


def _tc_towers(pooled_u, pooled_p, Wq1, bq1, Wq2, bq2, Wc1, bc1, Wc2, bc2):
  full = lambda shape: pl.BlockSpec(shape, lambda i: (0, 0))
  return pl.pallas_call(
      _tc_mlp_body,
      grid=(B // BM,),
      in_specs=[
          pl.BlockSpec((BM, D), lambda i: (i, 0)),
          pl.BlockSpec((BM, D), lambda i: (i, 0)),
          full((D, H)), full((1, H)), full((H, OUT)), full((1, OUT)),
          full((D, H)), full((1, H)), full((H, OUT)), full((1, OUT)),
      ],
      out_specs=[
          pl.BlockSpec((BM, OUT), lambda i: (i, 0)),
          pl.BlockSpec((BM, OUT), lambda i: (i, 0)),
      ],
      out_shape=[
          jax.ShapeDtypeStruct((B, OUT), jnp.float32),
          jax.ShapeDtypeStruct((B, OUT), jnp.float32),
      ],
  )(pooled_u, pooled_p,
    Wq1, bq1.reshape(1, H), Wq2, bq2.reshape(1, OUT),
    Wc1, bc1.reshape(1, H), Wc2, bc2.reshape(1, OUT))


@jax.jit
def kernel(user_ids, product_ids, user_table, product_table,
           Wq1, bq1, Wq2, bq2, Wc1, bc1, Wc2, bc2):
  pooled_u, pooled_p = _sc_gather_both(
      user_table, product_table, user_ids, product_ids)
  q, c = _tc_towers(pooled_u, pooled_p,
                    Wq1, bq1, Wq2, bq2, Wc1, bc1, Wc2, bc2)
  return (q, c)


# speedup vs baseline: 2.1294x; 1.0392x over previous
import functools

import jax
import jax.numpy as jnp
from jax import lax
from jax.experimental import pallas as pl
from jax.experimental.pallas import tpu as pltpu
from jax.experimental.pallas import tpu_sc as plsc

B = 16384
D = 64
H = 128
OUT = 64

NC = 2
NS = 16
NW = NC * NS
B_PER_W = B // NW

LAGG = 8
NCHUNK = 2
CH = B_PER_W // NCHUNK


def _sc_gather_both(user_table, product_table, user_ids, product_ids):
  mesh = plsc.VectorSubcoreMesh(core_axis_name="c", subcore_axis_name="s")

  @functools.partial(
      pl.kernel,
      mesh=mesh,
      compiler_params=pltpu.CompilerParams(disable_bounds_checks=True),
      out_type=(
          jax.ShapeDtypeStruct((B, D), jnp.float32),
          jax.ShapeDtypeStruct((B, D), jnp.float32),
      ),
      scratch_types=[
          pltpu.VMEM((B_PER_W,), jnp.int32),
          pltpu.VMEM((B_PER_W,), jnp.int32),
          pltpu.VMEM((CH, D), jnp.float32),
          pltpu.VMEM((CH, D), jnp.float32),
          pltpu.SemaphoreType.DMA,
      ],
  )
  def k(utab_hbm, ptab_hbm, uid_hbm, pid_hbm, uout_hbm, pout_hbm,
        uidx_s, pidx_s, urows, prows, sem):
    wid = lax.axis_index("s") * NC + lax.axis_index("c")
    base = wid * B_PER_W
    pltpu.sync_copy(uid_hbm.at[pl.ds(base, B_PER_W)], uidx_s)
    pltpu.sync_copy(pid_hbm.at[pl.ds(base, B_PER_W)], pidx_s)

    ng = CH // 16

    def fire_group(tab_hbm, idx_s, rows, coff, g):
      v = idx_s[pl.ds(coff + g * 16, 16)]
      for j in range(16):
        pltpu.async_copy(
            tab_hbm.at[pl.ds(v[j], 1)],
            rows.at[pl.ds(g * 16 + j, 1)], sem)

    def drain_groups():
      for _ in range(32):
        pltpu.make_async_copy(
            utab_hbm.at[pl.ds(0, 1)],
            urows.at[pl.ds(0, 1)], sem).wait()

    for c in range(NCHUNK):
      coff = c * CH

      @pl.loop(0, ng)
      def _body(g):
        fire_group(utab_hbm, uidx_s, urows, coff, g)
        fire_group(ptab_hbm, pidx_s, prows, coff, g)

        @pl.when(g >= LAGG)
        def _drain():
          drain_groups()

      @pl.loop(0, min(LAGG, ng))
      def _epilogue(g):
        drain_groups()

      pltpu.sync_copy(urows, uout_hbm.at[pl.ds(base + coff, CH)])
      pltpu.sync_copy(prows, pout_hbm.at[pl.ds(base + coff, CH)])

  return k(user_table, product_table, user_ids, product_ids)


BM = 2048


def _tc_mlp_body(u_ref, p_ref, wq1, bq1, wq2, bq2,
                 wc1, bc1, wc2, bc2, q_ref, c_ref):
  q = jnp.maximum(
      jnp.dot(u_ref[...], wq1[...], preferred_element_type=jnp.float32)
      + bq1[...], 0.0)
  q_ref[...] = jnp.maximum(
      jnp.dot(q, wq2[...], preferred_element_type=jnp.float32)
      + bq2[...], 0.0)
  c = jnp.maximum(
      jnp.dot(p_ref[...], wc1[...], preferred_element_type=jnp.float32)
      + bc1[...], 0.0)
  c_ref[...] = jnp.maximum(
      jnp.dot(c, wc2[...], preferred_element_type=jnp.float32)
      + bc2[...], 0.0)


def _tc_towers(pooled_u, pooled_p, Wq1, bq1, Wq2, bq2, Wc1, bc1, Wc2, bc2):
  full = lambda shape: pl.BlockSpec(shape, lambda i: (0, 0))
  return pl.pallas_call(
      _tc_mlp_body,
      grid=(B // BM,),
      in_specs=[
          pl.BlockSpec((BM, D), lambda i: (i, 0)),
          pl.BlockSpec((BM, D), lambda i: (i, 0)),
          full((D, H)), full((1, H)), full((H, OUT)), full((1, OUT)),
          full((D, H)), full((1, H)), full((H, OUT)), full((1, OUT)),
      ],
      out_specs=[
          pl.BlockSpec((BM, OUT), lambda i: (i, 0)),
          pl.BlockSpec((BM, OUT), lambda i: (i, 0)),
      ],
      out_shape=[
          jax.ShapeDtypeStruct((B, OUT), jnp.float32),
          jax.ShapeDtypeStruct((B, OUT), jnp.float32),
      ],
  )(pooled_u, pooled_p,
    Wq1, bq1.reshape(1, H), Wq2, bq2.reshape(1, OUT),
    Wc1, bc1.reshape(1, H), Wc2, bc2.reshape(1, OUT))


@jax.jit
def kernel(user_ids, product_ids, user_table, product_table,
           Wq1, bq1, Wq2, bq2, Wc1, bc1, Wc2, bc2):
  pooled_u, pooled_p = _sc_gather_both(
      user_table, product_table, user_ids, product_ids)
  q, c = _tc_towers(pooled_u, pooled_p,
                    Wq1, bq1, Wq2, bq2, Wc1, bc1, Wc2, bc2)
  return (q, c)
